# Initial kernel scaffold; baseline (speedup 1.0000x reference)
import jax
import jax.numpy as jnp
from jax.experimental import pallas as pl


def kernel(inputs, w):
    raise NotImplementedError("write your pallas kernel here")


# speedup vs baseline: 5.5388x; 5.5388x over previous
import functools

import jax
import jax.numpy as jnp
from jax import lax
from jax.experimental import pallas as pl
from jax.experimental.pallas import tpu as pltpu
from jax.experimental.pallas import tpu_sc as plsc

jax.config.update("jax_enable_x64", True)

_N = 1_000_000
_B = 16384
_LANES = 128
_PADDED = 7816 * _LANES
_PAD = _PADDED - _N
_NW = 32
_B_PER_W = _B // _NW
_CHUNKS = _B_PER_W // _LANES


@functools.cache
def _make_sc_gather():
    mesh = plsc.VectorSubcoreMesh(core_axis_name="c", subcore_axis_name="s")

    @functools.partial(
        pl.kernel,
        mesh=mesh,
        out_type=jax.ShapeDtypeStruct((_B,), jnp.float32),
        scratch_types=[
            pltpu.VMEM((_CHUNKS, _LANES), jnp.int32),
            pltpu.VMEM((_B_PER_W,), jnp.float32),
            pltpu.SemaphoreType.DMA,
        ],
    )
    def _sc_gather(w_hbm, idx_hbm, out_hbm, idx_v, g_v, sem):
        wid = lax.axis_index("s") * 2 + lax.axis_index("c")
        pltpu.sync_copy(idx_hbm.at[wid], idx_v)
        copies = [
            pltpu.async_copy(
                w_hbm.at[idx_v.at[jnp.int32(j)]],
                g_v.at[pl.ds(j * _LANES, _LANES)],
                sem,
            )
            for j in range(_CHUNKS)
        ]
        for c in copies:
            c.wait()
        pltpu.sync_copy(g_v, out_hbm.at[pl.ds(wid * _B_PER_W, _B_PER_W)])

    return _sc_gather


def _lse_body(x_ref, o_ref):
    x = x_ref[...]
    m = jnp.max(x)
    o_ref[0, 0] = m + jnp.log(jnp.sum(jnp.exp(x - m)))


_lse_call = pl.pallas_call(
    _lse_body,
    out_shape=jax.ShapeDtypeStruct((1, 1), jnp.float32),
    out_specs=pl.BlockSpec(memory_space=pltpu.SMEM),
)


def _sub_body(l_ref, g_ref, o_ref):
    o_ref[...] = g_ref[...] - l_ref[0, 0]


_sub_call = pl.pallas_call(
    _sub_body,
    out_shape=jax.ShapeDtypeStruct((_LANES, _LANES), jnp.float32),
    in_specs=[
        pl.BlockSpec(memory_space=pltpu.SMEM),
        pl.BlockSpec((_LANES, _LANES), lambda: (0, 0)),
    ],
)


def kernel(inputs, w):
    w32 = jnp.pad(w.astype(jnp.float32), (0, _PAD), constant_values=-jnp.inf)
    idx = inputs.astype(jnp.int32).reshape(_NW, _CHUNKS, _LANES)
    lse = _lse_call(w32.reshape(_PADDED // _LANES, _LANES))
    g = _make_sc_gather()(w32, idx)
    out = _sub_call(lse, g.reshape(_LANES, _LANES))
    return out.reshape(_B).astype(jnp.float64)
